# baseline (speedup 1.0000x reference)
import functools

import jax
import jax.numpy as jnp
from jax import lax
from jax.experimental import pallas as pl
from jax.experimental.pallas import tpu as pltpu
from jax.experimental.pallas import tpu_sc as plsc

_VOCAB = 1000000
_D = 32

_NC, _NS = 2, 16
_NW = _NC * _NS
_IDX_W = 128
_K = 8
_CHUNK = _IDX_W * _K
_G = 128 // _D
_LCHUNK = _CHUNK // _G


def _gather_rows(W, ids2d):
    n_rows = ids2d.shape[0] * _IDX_W
    rows_per_w = n_rows // _NW
    chunks_per_w = rows_per_w // _CHUNK
    idx_rows_per_w = ids2d.shape[0] // _NW
    lines_per_w = rows_per_w // _G

    mesh = plsc.VectorSubcoreMesh(core_axis_name="c", subcore_axis_name="s")

    @functools.partial(
        pl.kernel,
        out_type=jax.ShapeDtypeStruct((n_rows // _G, 128), jnp.float32),
        mesh=mesh,
        scratch_types=[
            pltpu.VMEM((_K, _IDX_W), jnp.int32),
            pltpu.VMEM((_CHUNK, _D), jnp.float32),
            pltpu.SemaphoreType.DMA,
        ],
        compiler_params=pltpu.CompilerParams(use_tc_tiling_on_sc=False),
    )
    def k(w_hbm, i_hbm, o_hbm, idx_v, rows_v, sem):
        wid = lax.axis_index("s") * _NC + lax.axis_index("c")
        idx_row0 = wid * idx_rows_per_w
        line0 = wid * lines_per_w

        @pl.loop(0, chunks_per_w)
        def _(c):
            pltpu.sync_copy(i_hbm.at[pl.ds(idx_row0 + c * _K, _K)], idx_v)
            copies = []
            for j in range(_K):
                copies.append(
                    pltpu.async_copy(
                        w_hbm.at[idx_v.at[j]],
                        rows_v.at[pl.ds(j * _IDX_W, _IDX_W)],
                        sem,
                    )
                )
            for cp in copies:
                cp.wait()
            for g in range(_G):
                pltpu.sync_copy(
                    rows_v.at[pl.ds(g * _LCHUNK, _LCHUNK)],
                    o_hbm.at[
                        pl.ds(line0 + c * _LCHUNK, _LCHUNK),
                        pl.ds(g * _D, _D),
                    ],
                )

    return k(W, ids2d)


_BB = 64


def _post_body(g_ref, o_ref):
    x = g_ref[...]
    li = lax.broadcasted_iota(jnp.int32, (128, 128), 0) // _D
    lj = lax.broadcasted_iota(jnp.int32, (128, 128), 1) // _D
    m = (li == lj).astype(jnp.float32)
    s = jax.lax.dot(x * x, m)
    o_ref[...] = x / jnp.maximum(jnp.sqrt(s), 1e-12)


def _normalize(G):
    n = G.shape[0]
    blk = _BB * 50 // _G
    return pl.pallas_call(
        _post_body,
        grid=(n // blk,),
        in_specs=[pl.BlockSpec((blk, 128), lambda i: (i, 0))],
        out_specs=pl.BlockSpec((blk, 128), lambda i: (i, 0)),
        out_shape=jax.ShapeDtypeStruct((n, 128), jnp.float32),
    )(G)


def kernel(species_ids, W):
    ids = jnp.clip(species_ids.astype(jnp.int32), 0, _VOCAB - 1)
    idsp = (
        ids.reshape(-1, _LCHUNK, _G)
        .transpose(0, 2, 1)
        .reshape(-1, _IDX_W)
    )
    lines = _gather_rows(W, idsp)
    normed = _normalize(lines)
    return normed.reshape(-1, _D).reshape(ids.shape + (_D,))


# speedup vs baseline: 1.0243x; 1.0243x over previous
import functools

import jax
import jax.numpy as jnp
from jax import lax
from jax.experimental import pallas as pl
from jax.experimental.pallas import tpu as pltpu
from jax.experimental.pallas import tpu_sc as plsc

_VOCAB = 1000000
_D = 32

_NC, _NS = 2, 16
_NW = _NC * _NS
_IDX_W = 128
_K = 8
_CHUNK = _IDX_W * _K
_G = 128 // _D
_LCHUNK = _CHUNK // _G


def _gather_rows(W, ids2d):
    n_rows = ids2d.shape[0] * _IDX_W
    rows_per_w = n_rows // _NW
    chunks_per_w = rows_per_w // _CHUNK
    idx_rows_per_w = ids2d.shape[0] // _NW
    lines_per_w = rows_per_w // _G

    mesh = plsc.VectorSubcoreMesh(core_axis_name="c", subcore_axis_name="s")

    @functools.partial(
        pl.kernel,
        out_type=jax.ShapeDtypeStruct((n_rows // _G, 128), jnp.float32),
        mesh=mesh,
        scratch_types=[
            pltpu.VMEM((_K, _IDX_W), jnp.int32),
            pltpu.VMEM((_CHUNK, _D), jnp.float32),
            pltpu.SemaphoreType.DMA,
        ],
        compiler_params=pltpu.CompilerParams(use_tc_tiling_on_sc=False),
    )
    def k(w_hbm, i_hbm, o_hbm, idx_v, rows_v, sem):
        wid = lax.axis_index("s") * _NC + lax.axis_index("c")
        idx_row0 = wid * idx_rows_per_w
        line0 = wid * lines_per_w

        @pl.loop(0, chunks_per_w)
        def _(c):
            pltpu.sync_copy(i_hbm.at[pl.ds(idx_row0 + c * _K, _K)], idx_v)
            copies = []
            for j in range(_K):
                copies.append(
                    pltpu.async_copy(
                        w_hbm.at[idx_v.at[j]],
                        rows_v.at[pl.ds(j * _IDX_W, _IDX_W)],
                        sem,
                    )
                )
            for cp in copies:
                cp.wait()
            for g in range(_G):
                pltpu.sync_copy(
                    rows_v.at[pl.ds(g * _LCHUNK, _LCHUNK)],
                    o_hbm.at[
                        pl.ds(line0 + c * _LCHUNK, _LCHUNK),
                        pl.ds(g * _D, _D),
                    ],
                )

    return k(W, ids2d)


_BB = 64


def _post_body(g_ref, o_ref):
    x = g_ref[...]
    li = lax.broadcasted_iota(jnp.int32, (128, 128), 0) // _D
    lj = lax.broadcasted_iota(jnp.int32, (128, 128), 1) // _D
    m = (li == lj).astype(jnp.float32)
    s = jax.lax.dot(x * x, m)
    o_ref[...] = x / jnp.maximum(jnp.sqrt(s), 1e-12)


def _normalize(G):
    n = G.shape[0]
    blk = _BB * 50 // _G
    return pl.pallas_call(
        _post_body,
        grid=(n // blk,),
        in_specs=[pl.BlockSpec((blk, 128), lambda i: (i, 0))],
        out_specs=pl.BlockSpec((blk, 128), lambda i: (i, 0)),
        out_shape=jax.ShapeDtypeStruct((n, 128), jnp.float32),
    )(G)


def kernel(species_ids, W):
    ids = jnp.clip(species_ids.astype(jnp.int32), 0, _VOCAB - 1)
    span = _G * _IDX_W
    x2c = ids.reshape(-1, _CHUNK)
    cols = [
        x2c[:, sp * span + g : (sp + 1) * span : _G]
        for g in range(_G)
        for sp in range(_CHUNK // span)
    ]
    idsp = jnp.stack(cols, axis=1).reshape(-1, _IDX_W)
    lines = _gather_rows(W, idsp)
    normed = _normalize(lines)
    return normed.reshape(-1, _D).reshape(ids.shape + (_D,))
